# baseline (speedup 1.0000x reference)
import functools

import jax
import jax.numpy as jnp
from jax import lax
from jax.experimental import pallas as pl
from jax.experimental.pallas import tpu as pltpu
from jax.experimental.pallas import tpu_sc as plsc

B, P, D, OUT = 4096, 32, 128, 128
NC, NS = 2, 16
NW = NC * NS
BPW = B // NW
BLK = 8
NBLK = BPW // BLK
LANES = 16
DC = D // LANES


NBUF = 3


def _seg_sum_body(x_hbm, out_hbm, buf0, buf1, buf2, out_stage,
                  sem0, sem1, sem2):
    c = lax.axis_index("c")
    s = lax.axis_index("s")
    wid = s * NC + c
    base = wid * BPW
    bufs = (buf0, buf1, buf2)
    sems = (sem0, sem1, sem2)

    for i in range(NBUF):
        pltpu.async_copy(x_hbm.at[pl.ds(base + i * BLK, BLK)], bufs[i],
                         sems[i])

    def group(gg, carry):
        for i in range(NBUF):
            g = gg * NBUF + i
            buf, sem = bufs[i], sems[i]
            pltpu.make_async_copy(x_hbm.at[pl.ds(0, BLK)], buf, sem).wait()

            def bbody(b, carry2):
                accs = [buf[b, 0, pl.ds(j * LANES, LANES)] for j in range(DC)]
                for p in range(1, P):
                    for j in range(DC):
                        accs[j] = accs[j] + buf[b, p, pl.ds(j * LANES, LANES)]
                row = g * BLK + b
                for j in range(DC):
                    out_stage[row, pl.ds(j * LANES, LANES)] = accs[j]
                return carry2

            lax.fori_loop(0, BLK, bbody, 0)

            nxt = g + NBUF

            @pl.when(nxt < NBLK)
            def _():
                pltpu.async_copy(
                    x_hbm.at[pl.ds(base + nxt * BLK, BLK)], buf, sem)
        return carry

    lax.fori_loop(0, NBLK // NBUF, group, 0)
    for g in range((NBLK // NBUF) * NBUF, NBLK):
        i = g % NBUF
        buf, sem = bufs[i], sems[i]
        pltpu.make_async_copy(x_hbm.at[pl.ds(0, BLK)], buf, sem).wait()

        def btail(b, carry2):
            accs = [buf[b, 0, pl.ds(j * LANES, LANES)] for j in range(DC)]
            for p in range(1, P):
                for j in range(DC):
                    accs[j] = accs[j] + buf[b, p, pl.ds(j * LANES, LANES)]
            row = g * BLK + b
            for j in range(DC):
                out_stage[row, pl.ds(j * LANES, LANES)] = accs[j]
            return carry2

        lax.fori_loop(0, BLK, btail, 0)
    pltpu.sync_copy(out_stage, out_hbm.at[pl.ds(base, BPW)])


_seg_sum = pl.kernel(
    _seg_sum_body,
    out_type=jax.ShapeDtypeStruct((B, D), jnp.float32),
    mesh=plsc.VectorSubcoreMesh(core_axis_name="c", subcore_axis_name="s"),
    scratch_types=[
        pltpu.VMEM((BLK, P, D), jnp.float32),
        pltpu.VMEM((BLK, P, D), jnp.float32),
        pltpu.VMEM((BLK, P, D), jnp.float32),
        pltpu.VMEM((BPW, D), jnp.float32),
        pltpu.SemaphoreType.DMA,
        pltpu.SemaphoreType.DMA,
        pltpu.SemaphoreType.DMA,
    ],
)


def _mm_body(s_ref, w_ref, b_ref, o_ref):
    o_ref[...] = lax.dot_general(
        s_ref[...], w_ref[...],
        (((1,), (1,)), ((), ())),
        preferred_element_type=jnp.float32,
    ) + b_ref[...] * float(P)


def _matmul(s, w, b2):
    return pl.pallas_call(
        _mm_body,
        out_shape=jax.ShapeDtypeStruct((B, OUT), jnp.float32),
    )(s, w, b2)


@jax.jit
def kernel(pert_batch, W, b):
    s = _seg_sum(pert_batch)
    return _matmul(s, W, b.reshape(1, OUT))
